# baseline (speedup 1.0000x reference)
import jax
import jax.numpy as jnp
from jax import lax
from jax.experimental import pallas as pl
from jax.experimental.pallas import tpu as pltpu
from jax.experimental.pallas import tpu_sc as plsc

VOCAB = 1000000
EMBED_DIM = 32
BATCH = 4096
HIST = 200

NUM_CORES = 2
NUM_SUBCORES = 16
NUM_WORKERS = NUM_CORES * NUM_SUBCORES

TOTAL = BATCH * HIST
PER_WORKER = TOTAL // NUM_WORKERS
CHUNK = 1280
NCHUNKS = PER_WORKER // CHUNK
BLK = 128
BLKS = CHUNK // BLK
BLOCKS_PER_WORKER = PER_WORKER // BLK
TC_GRID = BATCH // BLK


def _gather_body(table_hbm, idx_hbm, out_hbm, *scratch):
    idx_v = scratch[0:2]
    rows_v = scratch[2:4]
    trans_v = scratch[4:6]
    idx_sems = scratch[6:8]
    gat_sems = scratch[8:10]
    tst_sems = scratch[10:12]

    wid = lax.axis_index("s") * NUM_CORES + lax.axis_index("c")
    base = wid * PER_WORKER
    base_blk = wid * BLOCKS_PER_WORKER

    def idx_start(g, b):
        pltpu.make_async_copy(
            idx_hbm.at[pl.ds(base + g * CHUNK, CHUNK)], idx_v[b], idx_sems[b]
        ).start()

    def idx_wait(b):
        pltpu.make_async_copy(
            idx_hbm.at[pl.ds(base, CHUNK)], idx_v[b], idx_sems[b]
        ).wait()

    def gather_start(b):
        pltpu.make_async_copy(table_hbm.at[idx_v[b]], rows_v[b], gat_sems[b]).start()

    def gather_wait(b):
        pltpu.make_async_copy(table_hbm.at[idx_v[b]], rows_v[b], gat_sems[b]).wait()

    def tstore_drain(tb):
        for _ in range(4):
            pltpu.make_async_copy(
                trans_v[tb].at[pl.ds(0, 8), :], out_hbm.at[0, 0, 0], tst_sems[tb]
            ).wait()

    lanes = jax.lax.iota(jnp.int32, 16)

    def process(gm1, b):
        def blk_step(blk, carry):
            tb_sel = lax.rem(blk, 2)
            gblk = base_blk + gm1 * BLKS + blk
            h = lax.div(gblk, TC_GRID)
            tcb = lax.rem(gblk, TC_GRID)
            q0 = blk * BLK

            def per_buf(tb):
                @pl.when(blk >= 2)
                def _():
                    tstore_drain(tb)

                row_vecs = [q0 + 16 * k + lanes for k in range(BLK // 16)]
                for e in range(EMBED_DIM):
                    col = jnp.full((16,), e, jnp.int32)
                    for k in range(BLK // 16):
                        v = plsc.load_gather(rows_v[b], [row_vecs[k], col])
                        trans_v[tb][e, pl.ds(16 * k, 16)] = v

                for tr in range(4):
                    pltpu.make_async_copy(
                        trans_v[tb].at[pl.ds(8 * tr, 8), :],
                        out_hbm.at[h, tr, tcb],
                        tst_sems[tb],
                    ).start()

            @pl.when(tb_sel == 0)
            def _():
                per_buf(0)

            @pl.when(tb_sel == 1)
            def _():
                per_buf(1)

            return carry

        lax.fori_loop(0, BLKS, blk_step, 0)
        tstore_drain(0)
        tstore_drain(1)

    idx_start(0, 0)
    idx_start(1, 1)

    def outer_step(o, carry):
        idx_wait(0)
        gather_start(0)

        @pl.when(o > 0)
        def _():
            gather_wait(1)
            idx_start(2 * o + 1, 1)
            process(2 * o - 1, 1)

        idx_wait(1)
        gather_start(1)
        gather_wait(0)

        @pl.when(o < NCHUNKS // 2 - 1)
        def _():
            idx_start(2 * o + 2, 0)

        process(2 * o, 0)
        return carry

    lax.fori_loop(0, NCHUNKS // 2, outer_step, 0)

    gather_wait(1)
    process(NCHUNKS - 1, 1)


_mesh = plsc.VectorSubcoreMesh(core_axis_name="c", subcore_axis_name="s")

_scratch = (
    [pltpu.VMEM((CHUNK,), jnp.int32) for _ in range(2)]
    + [pltpu.VMEM((CHUNK, EMBED_DIM), jnp.float32) for _ in range(2)]
    + [pltpu.VMEM((EMBED_DIM, BLK), jnp.float32) for _ in range(2)]
    + [pltpu.SemaphoreType.DMA for _ in range(6)]
)

_gather = pl.kernel(
    _gather_body,
    out_type=jax.ShapeDtypeStruct((HIST, 4, TC_GRID, 8, BLK), jnp.float32),
    mesh=_mesh,
    scratch_types=_scratch,
    compiler_params=pltpu.CompilerParams(
        use_tc_tiling_on_sc=False, needs_layout_passes=False
    ),
)


@jax.jit
def kernel(indices, table):
    idx_hm = indices.T.reshape(TOTAL)
    out5 = _gather(table, idx_hm)
    return out5.transpose(2, 4, 0, 1, 3).reshape(BATCH, HIST, EMBED_DIM)


# speedup vs baseline: 1.1566x; 1.1566x over previous
import jax
import jax.numpy as jnp
from jax import lax
from jax.experimental import pallas as pl
from jax.experimental.pallas import tpu as pltpu
from jax.experimental.pallas import tpu_sc as plsc

VOCAB = 1000000
EMBED_DIM = 32
BATCH = 4096
HIST = 200

NUM_CORES = 2
NUM_SUBCORES = 16
NUM_WORKERS = NUM_CORES * NUM_SUBCORES

TOTAL = BATCH * HIST
PER_WORKER = TOTAL // NUM_WORKERS
CHUNK = 1280
NCHUNKS = PER_WORKER // CHUNK
NBUF = 2
OUTER = NCHUNKS // NBUF
NSTREAMS = 4
SUB = CHUNK // NSTREAMS


def _gather_body(table_hbm, idx_hbm, out_hbm, *scratch):
    idx_v = scratch[0:NBUF]
    rows_v = scratch[NBUF:2 * NBUF]
    idx_sems = scratch[2 * NBUF:3 * NBUF]
    gat_sems = scratch[3 * NBUF:3 * NBUF + NBUF * NSTREAMS]
    st_sems = scratch[3 * NBUF + NBUF * NSTREAMS:]

    wid = lax.axis_index("s") * NUM_CORES + lax.axis_index("c")
    base = wid * PER_WORKER

    def idx_start(g, b):
        pltpu.make_async_copy(
            idx_hbm.at[pl.ds(base + g * CHUNK, CHUNK)], idx_v[b], idx_sems[b]
        ).start()

    def idx_wait(b):
        pltpu.make_async_copy(
            idx_hbm.at[pl.ds(base, CHUNK)], idx_v[b], idx_sems[b]
        ).wait()

    def gather_start(b):
        for s in range(NSTREAMS):
            pltpu.make_async_copy(
                table_hbm.at[idx_v[b].at[pl.ds(s * SUB, SUB)]],
                rows_v[b].at[pl.ds(s * SUB, SUB)],
                gat_sems[b * NSTREAMS + s],
            ).start()

    def gather_wait(b):
        for s in range(NSTREAMS):
            pltpu.make_async_copy(
                table_hbm.at[idx_v[b].at[pl.ds(s * SUB, SUB)]],
                rows_v[b].at[pl.ds(s * SUB, SUB)],
                gat_sems[b * NSTREAMS + s],
            ).wait()

    def store_start(g, b):
        pltpu.make_async_copy(
            rows_v[b], out_hbm.at[pl.ds(base + g * CHUNK, CHUNK)], st_sems[b]
        ).start()

    def store_wait(b):
        pltpu.make_async_copy(
            rows_v[b], out_hbm.at[pl.ds(base, CHUNK)], st_sems[b]
        ).wait()

    for b in range(NBUF):
        idx_start(b, b)

    def outer_step(o, carry):
        for b in range(NBUF):
            g = o * NBUF + b
            def retire(c, cb):
                gather_wait(cb)
                store_start(c, cb)

                @pl.when(c + NBUF < NCHUNKS)
                def _():
                    idx_start(c + NBUF, cb)

            if b == 0:
                @pl.when(o > 0)
                def _():
                    retire(g - 1, NBUF - 1)
            else:
                retire(g - 1, b - 1)

            @pl.when(o > 0)
            def _():
                store_wait(b)

            idx_wait(b)
            gather_start(b)
        return carry

    lax.fori_loop(0, OUTER, outer_step, 0)

    last_b = (NCHUNKS - 1) % NBUF
    gather_wait(last_b)
    store_start(NCHUNKS - 1, last_b)
    for b in range(NBUF):
        store_wait(b)


_mesh = plsc.VectorSubcoreMesh(core_axis_name="c", subcore_axis_name="s")

_scratch = (
    [pltpu.VMEM((CHUNK,), jnp.int32) for _ in range(NBUF)]
    + [pltpu.VMEM((CHUNK, EMBED_DIM), jnp.float32) for _ in range(NBUF)]
    + [pltpu.SemaphoreType.DMA for _ in range(2 * NBUF + NBUF * NSTREAMS)]
)

_gather = pl.kernel(
    _gather_body,
    out_type=jax.ShapeDtypeStruct((TOTAL, EMBED_DIM), jnp.float32),
    mesh=_mesh,
    scratch_types=_scratch,
    compiler_params=pltpu.CompilerParams(use_tc_tiling_on_sc=False),
)


@jax.jit
def kernel(indices, table):
    flat = indices.reshape(TOTAL)
    out = _gather(table, flat)
    return out.reshape(BATCH, HIST, EMBED_DIM)


# speedup vs baseline: 1.1575x; 1.0008x over previous
import jax
import jax.numpy as jnp
from jax import lax
from jax.experimental import pallas as pl
from jax.experimental.pallas import tpu as pltpu
from jax.experimental.pallas import tpu_sc as plsc

VOCAB = 1000000
EMBED_DIM = 32
BATCH = 4096
HIST = 200

NUM_CORES = 2
NUM_SUBCORES = 16
NUM_WORKERS = NUM_CORES * NUM_SUBCORES

TOTAL = BATCH * HIST
PER_WORKER = TOTAL // NUM_WORKERS
CHUNK = 1280
NCHUNKS = PER_WORKER // CHUNK
NBUF = 2
OUTER = NCHUNKS // NBUF
NSTREAMS = 4
SUB = CHUNK // NSTREAMS


def _gather_body(table_hbm, idx_hbm, out_hbm, *scratch):
    idx_v = scratch[0:NBUF]
    rows_v = scratch[NBUF:2 * NBUF]
    idx_sems = scratch[2 * NBUF:3 * NBUF]
    gat_sems = scratch[3 * NBUF:3 * NBUF + NBUF * NSTREAMS]
    st_sems = scratch[3 * NBUF + NBUF * NSTREAMS:]

    wid = lax.axis_index("s") * NUM_CORES + lax.axis_index("c")
    base = wid * PER_WORKER

    def idx_start(g, b):
        pltpu.make_async_copy(
            idx_hbm.at[pl.ds(base + g * CHUNK, CHUNK)], idx_v[b], idx_sems[b]
        ).start()

    def idx_wait(b):
        pltpu.make_async_copy(
            idx_hbm.at[pl.ds(base, CHUNK)], idx_v[b], idx_sems[b]
        ).wait()

    def gather_start(b):
        for s in range(NSTREAMS):
            pltpu.make_async_copy(
                table_hbm.at[idx_v[b].at[pl.ds(s * SUB, SUB)]],
                rows_v[b].at[pl.ds(s * SUB, SUB)],
                gat_sems[b * NSTREAMS + s],
            ).start()

    def gather_wait(b):
        for s in range(NSTREAMS):
            pltpu.make_async_copy(
                table_hbm.at[idx_v[b].at[pl.ds(s * SUB, SUB)]],
                rows_v[b].at[pl.ds(s * SUB, SUB)],
                gat_sems[b * NSTREAMS + s],
            ).wait()

    def store_start(g, b):
        pltpu.make_async_copy(
            rows_v[b], out_hbm.at[pl.ds(base + g * CHUNK, CHUNK)], st_sems[b]
        ).start()

    def store_wait(b):
        pltpu.make_async_copy(
            rows_v[b], out_hbm.at[pl.ds(base, CHUNK)], st_sems[b]
        ).wait()

    for b in range(NBUF):
        idx_start(b, b)

    def outer_step(o, carry):
        for b in range(NBUF):
            g = o * NBUF + b
            def retire(c, cb):
                gather_wait(cb)
                store_start(c, cb)

                @pl.when(c + NBUF < NCHUNKS)
                def _():
                    idx_start(c + NBUF, cb)

            if b == 0:
                @pl.when(o > 0)
                def _():
                    retire(g - 1, NBUF - 1)
            else:
                retire(g - 1, b - 1)

            @pl.when(o > 0)
            def _():
                store_wait(b)

            idx_wait(b)
            gather_start(b)
        return carry

    lax.fori_loop(0, OUTER, outer_step, 0)

    last_b = (NCHUNKS - 1) % NBUF
    gather_wait(last_b)
    store_start(NCHUNKS - 1, last_b)
    for b in range(NBUF):
        store_wait(b)


_mesh = plsc.VectorSubcoreMesh(core_axis_name="c", subcore_axis_name="s")

_scratch = (
    [pltpu.VMEM((CHUNK,), jnp.int32) for _ in range(NBUF)]
    + [pltpu.VMEM((CHUNK, EMBED_DIM), jnp.float32) for _ in range(NBUF)]
    + [pltpu.SemaphoreType.DMA for _ in range(2 * NBUF + NBUF * NSTREAMS)]
)

_gather = pl.kernel(
    _gather_body,
    out_type=jax.ShapeDtypeStruct((TOTAL, EMBED_DIM), jnp.float32),
    mesh=_mesh,
    scratch_types=_scratch,
    compiler_params=pltpu.CompilerParams(
        use_tc_tiling_on_sc=False, skip_device_barrier=True
    ),
)


@jax.jit
def kernel(indices, table):
    flat = indices.reshape(TOTAL)
    out = _gather(table, flat)
    return out.reshape(BATCH, HIST, EMBED_DIM)
